# baseline (speedup 1.0000x reference)
import functools

import jax
import jax.numpy as jnp
from jax import lax
from jax.experimental import pallas as pl
from jax.experimental.pallas import tpu as pltpu
from jax.experimental.pallas import tpu_sc as plsc

_NC, _NS, _L = 2, 16, 16
_NW = _NC * _NS


@functools.lru_cache(maxsize=None)
def _make_sc_kernel(N, D, E):
    EW = E // _NW
    C = 80
    n_chunks = EW // C
    n_groups = C // _L
    n_pairs = (n_chunks - 1) // 2
    assert EW % C == 0 and C % _L == 0 and D % _L == 0 and E % _NW == 0
    assert n_chunks == 2 * n_pairs + 1

    mesh = plsc.VectorSubcoreMesh(core_axis_name="c", subcore_axis_name="s")

    @functools.partial(
        pl.kernel,
        mesh=mesh,
        compiler_params=pltpu.CompilerParams(needs_layout_passes=False),
        out_type=jax.ShapeDtypeStruct((E,), jnp.float32),
        scratch_types=[
            pltpu.VMEM((EW,), jnp.int32),
            pltpu.VMEM((EW,), jnp.int32),
            pltpu.VMEM((C, D), jnp.int32),
            pltpu.VMEM((C, D), jnp.int32),
            pltpu.VMEM((C, D), jnp.int32),
            pltpu.VMEM((C, D), jnp.int32),
            pltpu.VMEM((EW,), jnp.float32),
            pltpu.VMEM((_L, _L), jnp.float32),
            pltpu.VMEM((_L, _L), jnp.float32),
            pltpu.SemaphoreType.DMA,
            pltpu.SemaphoreType.DMA,
            pltpu.SemaphoreType.DMA,
            pltpu.SemaphoreType.DMA,
        ],
    )
    def k(h_hbm, src_hbm, dst_hbm, out_hbm, isrc, idst, a0, b0, a1, b1,
          out_w, tt, tt2, sa0, sb0, sa1, sb1):
        wid = lax.axis_index("s") * _NC + lax.axis_index("c")
        base = wid * EW
        pltpu.sync_copy(src_hbm.at[pl.ds(base, EW)], isrc)
        pltpu.sync_copy(dst_hbm.at[pl.ds(base, EW)], idst)

        lane = jnp.arange(_L, dtype=jnp.int32)

        def issue(c, ba, bb, sema, semb):
            off = c * C
            pltpu.async_copy(h_hbm.at[isrc.at[pl.ds(off, C)]], ba, sema)
            pltpu.async_copy(h_hbm.at[idst.at[pl.ds(off, C)]], bb, semb)

        def wait_pair(ba, bb, sema, semb):
            pltpu.make_async_copy(h_hbm.at[pl.ds(0, C)], ba, sema).wait()
            pltpu.make_async_copy(h_hbm.at[pl.ds(0, C)], bb, semb).wait()

        def compute(c, ba, bb):
            off = c * C

            for g in range(n_groups):
                eb = g * _L
                tile = tt if g % 2 == 0 else tt2
                for j in range(_L):
                    acc = None
                    for kk in range(D // (2 * _L)):
                        va = plsc.bitcast(ba[eb + j, pl.ds(kk * _L, _L)],
                                          jnp.bfloat16)
                        vb = plsc.bitcast(bb[eb + j, pl.ds(kk * _L, _L)],
                                          jnp.bfloat16)
                        p = va * vb
                        acc = p if acc is None else acc + p
                    lo, hi = plsc.unpack(
                        acc, format=plsc.PackFormat.INTERLEAVED,
                        preferred_element_type=jnp.float32)
                    tile[j, pl.ds(0, _L)] = lo + hi
                cols = [plsc.load_gather(
                            tile, [lane, jnp.full((_L,), l, jnp.int32)])
                        for l in range(_L)]
                while len(cols) > 1:
                    cols = [cols[i] + cols[i + 1]
                            for i in range(0, len(cols), 2)]
                out_w[pl.ds(off + eb, _L)] = cols[0]

        issue(0, a0, b0, sa0, sb0)

        def pair_body(i, carry):
            c0 = 2 * i
            issue(c0 + 1, a1, b1, sa1, sb1)
            wait_pair(a0, b0, sa0, sb0)
            compute(c0, a0, b0)
            issue(c0 + 2, a0, b0, sa0, sb0)
            wait_pair(a1, b1, sa1, sb1)
            compute(c0 + 1, a1, b1)
            return carry

        lax.fori_loop(0, n_pairs, pair_body, 0)
        wait_pair(a0, b0, sa0, sb0)
        compute(n_chunks - 1, a0, b0)

        pltpu.sync_copy(out_w, out_hbm.at[pl.ds(base, EW)])

    return k


def kernel(h, edge_index):
    N, D = h.shape
    E = edge_index.shape[1]
    src = edge_index[0]
    dst = edge_index[1]
    hb = jax.lax.bitcast_convert_type(
        h.astype(jnp.bfloat16).reshape(N, D // 2, 2), jnp.int32)
    hb = jnp.concatenate([hb, jnp.zeros((N, D - D // 2), jnp.int32)], axis=1)
    out = _make_sc_kernel(N, D, E)(hb, src, dst)
    return out.reshape(E, 1)


# speedup vs baseline: 1.2185x; 1.2185x over previous
import functools

import jax
import jax.numpy as jnp
import numpy as np
from jax import lax
from jax.experimental import pallas as pl
from jax.experimental.pallas import tpu as pltpu
from jax.experimental.pallas import tpu_sc as plsc

_NC, _NS, _L = 2, 16, 16
_NW = _NC * _NS


@functools.lru_cache(maxsize=None)
def _make_sc_kernel(N, D, E):
    EW = E // _NW
    C = 80
    n_chunks = EW // C
    n_groups = C // _L
    n_pairs = (n_chunks - 1) // 2
    assert EW % C == 0 and C % _L == 0 and D % _L == 0 and E % _NW == 0
    assert n_chunks == 2 * n_pairs + 1

    mesh = plsc.VectorSubcoreMesh(core_axis_name="c", subcore_axis_name="s")

    @functools.partial(
        pl.kernel,
        mesh=mesh,
        compiler_params=pltpu.CompilerParams(needs_layout_passes=False),
        out_type=jax.ShapeDtypeStruct((E,), jnp.float32),
        scratch_types=[
            pltpu.VMEM((EW,), jnp.int32),
            pltpu.VMEM((EW,), jnp.int32),
            pltpu.VMEM((C, D), jnp.int32),
            pltpu.VMEM((C, D), jnp.int32),
            pltpu.VMEM((C, D), jnp.int32),
            pltpu.VMEM((C, D), jnp.int32),
            pltpu.VMEM((EW,), jnp.float32),
            pltpu.VMEM((_L, _L), jnp.float32),
            pltpu.VMEM((_L, _L), jnp.float32),
            pltpu.SemaphoreType.DMA,
            pltpu.SemaphoreType.DMA,
            pltpu.SemaphoreType.DMA,
            pltpu.SemaphoreType.DMA,
        ],
    )
    def k(h_hbm, src_hbm, dst_hbm, out_hbm, isrc, idst, a0, b0, a1, b1,
          out_w, tt, tt2, sa0, sb0, sa1, sb1):
        wid = lax.axis_index("s") * _NC + lax.axis_index("c")
        base = wid * EW
        pltpu.sync_copy(src_hbm.at[pl.ds(base, EW)], isrc)
        pltpu.sync_copy(dst_hbm.at[pl.ds(base, EW)], idst)

        lane = jnp.arange(_L, dtype=jnp.int32)
        rots = [(lane + j) & (_L - 1) for j in range(_L)]
        rows = [jnp.full((_L,), j, jnp.int32) for j in range(_L)]

        def issue(c, ba, bb, sema, semb):
            off = c * C
            pltpu.async_copy(h_hbm.at[isrc.at[pl.ds(off, C)]], ba, sema)
            pltpu.async_copy(h_hbm.at[idst.at[pl.ds(off, C)]], bb, semb)

        def wait_pair(ba, bb, sema, semb):
            pltpu.make_async_copy(h_hbm.at[pl.ds(0, C)], ba, sema).wait()
            pltpu.make_async_copy(h_hbm.at[pl.ds(0, C)], bb, semb).wait()

        def compute(c, ba, bb):
            off = c * C

            def group_body(g, carry):
                eb = g * _L
                for j in range(_L):
                    acc = None
                    for kk in range(D // (2 * _L)):
                        va = plsc.bitcast(ba[eb + j, pl.ds(kk * _L, _L)],
                                          jnp.bfloat16)
                        vb = plsc.bitcast(bb[eb + j, pl.ds(kk * _L, _L)],
                                          jnp.bfloat16)
                        p = va * vb
                        acc = p if acc is None else acc + p
                    lo, hi = plsc.unpack(
                        acc, format=plsc.PackFormat.INTERLEAVED,
                        preferred_element_type=jnp.float32)
                    plsc.store_scatter(tt, [rows[j], rots[j]], lo + hi)
                cols = [plsc.load_gather(tt, [lane, rots[l]])
                        for l in range(_L)]
                while len(cols) > 1:
                    cols = [cols[i] + cols[i + 1]
                            for i in range(0, len(cols), 2)]
                out_w[pl.ds(off + eb, _L)] = cols[0]
                return carry

            lax.fori_loop(0, n_groups, group_body, 0)

        issue(0, a0, b0, sa0, sb0)

        def pair_body(i, carry):
            c0 = 2 * i
            issue(c0 + 1, a1, b1, sa1, sb1)
            wait_pair(a0, b0, sa0, sb0)
            compute(c0, a0, b0)
            issue(c0 + 2, a0, b0, sa0, sb0)
            wait_pair(a1, b1, sa1, sb1)
            compute(c0 + 1, a1, b1)
            return carry

        lax.fori_loop(0, n_pairs, pair_body, 0)
        wait_pair(a0, b0, sa0, sb0)
        compute(n_chunks - 1, a0, b0)

        pltpu.sync_copy(out_w, out_hbm.at[pl.ds(base, EW)])

    return k


def kernel(h, edge_index):
    N, D = h.shape
    E = edge_index.shape[1]
    src = edge_index[0]
    dst = edge_index[1]
    hb = jax.lax.bitcast_convert_type(
        h.astype(jnp.bfloat16).reshape(N, D // 2, 2), jnp.int32)
    hb = jnp.concatenate([hb, jnp.zeros((N, D - D // 2), jnp.int32)], axis=1)
    out = _make_sc_kernel(N, D, E)(hb, src, dst)
    return out.reshape(E, 1)


# speedup vs baseline: 1.2561x; 1.0309x over previous
import functools

import jax
import jax.numpy as jnp
import numpy as np
from jax import lax
from jax.experimental import pallas as pl
from jax.experimental.pallas import tpu as pltpu
from jax.experimental.pallas import tpu_sc as plsc

_NC, _NS, _L = 2, 16, 16
_NW = _NC * _NS


@functools.lru_cache(maxsize=None)
def _make_sc_kernel(N, D, E):
    EW = E // _NW
    C = 80
    n_chunks = EW // C
    n_groups = C // _L
    n_pairs = (n_chunks - 1) // 2
    assert EW % C == 0 and C % _L == 0 and D % _L == 0 and E % _NW == 0
    assert n_chunks == 2 * n_pairs + 1

    mesh = plsc.VectorSubcoreMesh(core_axis_name="c", subcore_axis_name="s")

    @functools.partial(
        pl.kernel,
        mesh=mesh,
        compiler_params=pltpu.CompilerParams(needs_layout_passes=False),
        out_type=jax.ShapeDtypeStruct((E,), jnp.float32),
        scratch_types=[
            pltpu.VMEM((EW,), jnp.int32),
            pltpu.VMEM((EW,), jnp.int32),
            pltpu.VMEM((C, D), jnp.int32),
            pltpu.VMEM((C, D), jnp.int32),
            pltpu.VMEM((C, D), jnp.int32),
            pltpu.VMEM((C, D), jnp.int32),
            pltpu.VMEM((EW,), jnp.float32),
            pltpu.VMEM((_L, _L), jnp.int32),
            pltpu.SemaphoreType.DMA,
            pltpu.SemaphoreType.DMA,
            pltpu.SemaphoreType.DMA,
            pltpu.SemaphoreType.DMA,
        ],
    )
    def k(h_hbm, src_hbm, dst_hbm, out_hbm, isrc, idst, a0, b0, a1, b1,
          out_w, tt, sa0, sb0, sa1, sb1):
        wid = lax.axis_index("s") * _NC + lax.axis_index("c")
        base = wid * EW
        pltpu.sync_copy(src_hbm.at[pl.ds(base, EW)], isrc)
        pltpu.sync_copy(dst_hbm.at[pl.ds(base, EW)], idst)

        lane = jnp.arange(_L, dtype=jnp.int32)
        rots = [(lane + j) & (_L - 1) for j in range(_L)]
        rows = [jnp.full((_L,), j, jnp.int32) for j in range(_L)]

        def issue(c, ba, bb, sema, semb):
            off = c * C
            pltpu.async_copy(h_hbm.at[isrc.at[pl.ds(off, C)]], ba, sema)
            pltpu.async_copy(h_hbm.at[idst.at[pl.ds(off, C)]], bb, semb)

        def wait_pair(ba, bb, sema, semb):
            pltpu.make_async_copy(h_hbm.at[pl.ds(0, C)], ba, sema).wait()
            pltpu.make_async_copy(h_hbm.at[pl.ds(0, C)], bb, semb).wait()

        def compute(c, ba, bb):
            off = c * C

            def group_body(g, carry):
                eb = g * _L
                for j in range(_L):
                    acc = None
                    for kk in range(D // (2 * _L)):
                        va = plsc.bitcast(ba[eb + j, pl.ds(kk * _L, _L)],
                                          jnp.bfloat16)
                        vb = plsc.bitcast(bb[eb + j, pl.ds(kk * _L, _L)],
                                          jnp.bfloat16)
                        p = va * vb
                        acc = p if acc is None else acc + p
                    plsc.store_scatter(tt, [rows[j], rots[j]],
                                       plsc.bitcast(acc, jnp.int32))
                cols = [plsc.bitcast(plsc.load_gather(tt, [lane, rots[l]]),
                                     jnp.bfloat16)
                        for l in range(_L)]
                while len(cols) > 1:
                    cols = [cols[i] + cols[i + 1]
                            for i in range(0, len(cols), 2)]
                lo, hi = plsc.unpack(
                    cols[0], format=plsc.PackFormat.INTERLEAVED,
                    preferred_element_type=jnp.float32)
                out_w[pl.ds(off + eb, _L)] = lo + hi
                return carry

            lax.fori_loop(0, n_groups, group_body, 0)

        issue(0, a0, b0, sa0, sb0)

        def pair_body(i, carry):
            c0 = 2 * i
            issue(c0 + 1, a1, b1, sa1, sb1)
            wait_pair(a0, b0, sa0, sb0)
            compute(c0, a0, b0)
            issue(c0 + 2, a0, b0, sa0, sb0)
            wait_pair(a1, b1, sa1, sb1)
            compute(c0 + 1, a1, b1)
            return carry

        lax.fori_loop(0, n_pairs, pair_body, 0)
        wait_pair(a0, b0, sa0, sb0)
        compute(n_chunks - 1, a0, b0)

        pltpu.sync_copy(out_w, out_hbm.at[pl.ds(base, EW)])

    return k


def kernel(h, edge_index):
    N, D = h.shape
    E = edge_index.shape[1]
    src = edge_index[0]
    dst = edge_index[1]
    hb = jax.lax.bitcast_convert_type(
        h.astype(jnp.bfloat16).reshape(N, D // 2, 2), jnp.int32)
    hb = jnp.concatenate([hb, jnp.zeros((N, D - D // 2), jnp.int32)], axis=1)
    out = _make_sc_kernel(N, D, E)(hb, src, dst)
    return out.reshape(E, 1)
